# baseline (speedup 1.0000x reference)
import functools

import jax
import jax.numpy as jnp
from jax import lax
from jax.experimental import pallas as pl
from jax.experimental.pallas import tpu as pltpu
from jax.experimental.pallas import tpu_sc as plsc

N = 10000
D = 128
E = 320000

NC = 1
NS = 16
NW = NC * NS

CHUNK = 128
NCH = 160
G = 16
EPT = NCH * CHUNK
E_PAD = NW * EPT
N_PAD = 10240
ROWS_PT = N_PAD // NS


def _sc_segment_sum_body(h_hbm, src_hbm, dst_hbm, out_hbm,
                         sidx0, sidx1, didx0, didx1, r0, r1, acc,
                         semA, semB, sem_slab):
    cid = lax.axis_index("c")
    sid = lax.axis_index("s")
    wid = cid * NS + sid

    def _zrow(i, carry):
        r0[i // 8, pl.ds((i % 8) * 16, 16)] = jnp.zeros((16,), jnp.float32)
        return carry

    lax.fori_loop(0, CHUNK * 8, _zrow, 0)

    def _zcp(k, carry):
        pltpu.sync_copy(r0, acc.at[pl.ds(sid * ROWS_PT + k * CHUNK, CHUNK)])
        return carry

    lax.fori_loop(0, ROWS_PT // CHUNK, _zcp, 0)
    plsc.subcore_barrier()

    NGRP = NCH // G
    slabs = [(sidx0, didx0), (sidx1, didx1)]
    pend = None
    for g in range(NGRP):
        sbuf, dbuf = slabs[g % 2]
        if g == 0:
            pltpu.sync_copy(src_hbm.at[wid, pl.ds(0, G)], sbuf)
            pltpu.sync_copy(dst_hbm.at[wid, pl.ds(0, G)], dbuf)
        else:
            for c in pend:
                c.wait()
        if g + 1 < NGRP:
            nsb, ndb = slabs[(g + 1) % 2]
            pend = (
                pltpu.async_copy(src_hbm.at[wid, pl.ds((g + 1) * G, G)],
                                 nsb, sem_slab),
                pltpu.async_copy(dst_hbm.at[wid, pl.ds((g + 1) * G, G)],
                                 ndb, sem_slab),
            )

        pltpu.async_copy(h_hbm.at[sbuf.at[0]], r0, semA)
        pltpu.async_copy(h_hbm.at[sbuf.at[1]], r1, semB)

        def _pair(s, carry, sbuf=sbuf, dbuf=dbuf):
            pltpu.make_async_copy(h_hbm.at[sbuf.at[2 * s]], r0, semA).wait()
            pltpu.sync_copy(r0, acc.at[dbuf.at[2 * s]], add=True)

            @pl.when(s < G // 2 - 1)
            def _():
                pltpu.async_copy(h_hbm.at[sbuf.at[2 * s + 2]], r0, semA)

            pltpu.make_async_copy(h_hbm.at[sbuf.at[2 * s + 1]], r1, semB).wait()
            pltpu.sync_copy(r1, acc.at[dbuf.at[2 * s + 1]], add=True)

            @pl.when(s < G // 2 - 1)
            def _():
                pltpu.async_copy(h_hbm.at[sbuf.at[2 * s + 3]], r1, semB)

            return carry

        lax.fori_loop(0, G // 2, _pair, 0)

    plsc.subcore_barrier()

    pltpu.sync_copy(acc.at[pl.ds(sid * ROWS_PT, ROWS_PT)],
                    out_hbm.at[cid, pl.ds(sid * ROWS_PT, ROWS_PT)])


_sc_segment_sum = pl.kernel(
    _sc_segment_sum_body,
    out_type=jax.ShapeDtypeStruct((NC, N_PAD, D), jnp.float32),
    mesh=plsc.VectorSubcoreMesh(core_axis_name="c", subcore_axis_name="s",
                                num_cores=NC),
    scratch_types=[
        pltpu.VMEM((G, CHUNK), jnp.int32),
        pltpu.VMEM((G, CHUNK), jnp.int32),
        pltpu.VMEM((G, CHUNK), jnp.int32),
        pltpu.VMEM((G, CHUNK), jnp.int32),
        pltpu.VMEM((CHUNK, D), jnp.float32),
        pltpu.VMEM((CHUNK, D), jnp.float32),
        pltpu.VMEM_SHARED((N_PAD, D), jnp.float32),
        pltpu.SemaphoreType.DMA,
        pltpu.SemaphoreType.DMA,
        pltpu.SemaphoreType.DMA,
    ],
)


BN = 512


def _mlp_body(last, h_ref, p0_ref, w1_ref, b1_ref, w2_ref, b2_ref, o_ref):
    m = h_ref[...] + p0_ref[...]
    t = jnp.dot(m, w1_ref[...], preferred_element_type=jnp.float32,
                precision=lax.Precision.HIGHEST) + b1_ref[...]
    t = jnp.maximum(t, 0.0)
    o = jnp.dot(t, w2_ref[...], preferred_element_type=jnp.float32,
                precision=lax.Precision.HIGHEST) + b2_ref[...]
    if last:
        nrm = jnp.sqrt(jnp.sum(o * o, axis=1, keepdims=True))
        o = o / jnp.maximum(nrm, 1e-12)
    else:
        o = jnp.maximum(o, 0.0)
    o_ref[...] = o


def _mlp(h, p0, w1, b1, w2, b2, last):
    row = pl.BlockSpec((BN, D), lambda i: (i, 0))
    full = pl.BlockSpec((D, D), lambda i: (0, 0))
    bias = pl.BlockSpec((1, D), lambda i: (0, 0))
    return pl.pallas_call(
        functools.partial(_mlp_body, last),
        grid=(N_PAD // BN,),
        in_specs=[row, row, full, bias, full, bias],
        out_specs=row,
        out_shape=jax.ShapeDtypeStruct((N_PAD, D), jnp.float32),
    )(h, p0, w1, b1.reshape(1, D), w2, b2.reshape(1, D))


def kernel(x, edge_index, edge_weight, W1_0, b1_0, W2_0, b2_0,
           W1_1, b1_1, W2_1, b2_1):
    pad = E_PAD - E
    src3 = jnp.concatenate([edge_index[0], jnp.zeros((pad,), jnp.int32)]
                           ).reshape(NW, NCH, CHUNK)
    dst3 = jnp.concatenate([edge_index[1], jnp.full((pad,), N, jnp.int32)]
                           ).reshape(NW, NCH, CHUNK)
    x_pad = jnp.zeros((N_PAD, D), jnp.float32).at[:N].set(x)

    parts = _sc_segment_sum(x_pad, src3, dst3)
    h1 = _mlp(x_pad, parts[0], W1_0, b1_0, W2_0, b2_0, last=False)
    parts = _sc_segment_sum(h1, src3, dst3)
    h2 = _mlp(h1, parts[0], W1_1, b1_1, W2_1, b2_1, last=True)
    return h2[:N]


# speedup vs baseline: 1.3949x; 1.3949x over previous
import functools

import jax
import jax.numpy as jnp
import numpy as np
from jax import lax
from jax.experimental import pallas as pl
from jax.experimental.pallas import tpu as pltpu
from jax.experimental.pallas import tpu_sc as plsc

N = 10000
D = 128
E = 320000

NC = 2
NS = 16
NW = NC * NS

CHUNK = 128
HALF = CHUNK // 2
DP = D // 2
NCH = 80
G = 16
G2 = 2 * G
EPT = NCH * CHUNK
E_PAD = NW * EPT
N_PAD = 10240
ROWS_PT = N_PAD // NS

PERM = np.concatenate([np.arange(0, D, 2), np.arange(1, D, 2)])


def _sc_segment_sum_body(h_hbm, src_hbm, dst_hbm, out_hbm,
                         sidx0, sidx1, didx0, didx1, p0, p1, f0, f1, acc,
                         semA, semB, sem_slab):
    cid = lax.axis_index("c")
    sid = lax.axis_index("s")
    wid = cid * NS + sid

    def _zrow(i, carry):
        f0[i // 8, pl.ds((i % 8) * 16, 16)] = jnp.zeros((16,), jnp.float32)
        return carry

    lax.fori_loop(0, HALF * 8, _zrow, 0)

    def _zcp(k, carry):
        pltpu.sync_copy(f0, acc.at[pl.ds(sid * ROWS_PT + k * HALF, HALF)])
        return carry

    lax.fori_loop(0, ROWS_PT // HALF, _zcp, 0)
    plsc.subcore_barrier()

    shift16 = jnp.full((16,), 16, jnp.int32)
    himask = jnp.full((16,), -65536, jnp.int32)

    def _convert(pbuf, fbuf, base):
        def _row(r, carry):
            for t in range(4):
                w = pbuf[base + r, pl.ds(16 * t, 16)]
                fbuf[r, pl.ds(16 * t, 16)] = plsc.bitcast(
                    lax.shift_left(w, shift16), jnp.float32)
                fbuf[r, pl.ds(64 + 16 * t, 16)] = plsc.bitcast(
                    lax.bitwise_and(w, himask), jnp.float32)
            return carry

        lax.fori_loop(0, HALF, _row, 0)

    NGRP = NCH // G
    slabs = [(sidx0, didx0), (sidx1, didx1)]
    pend = None
    for g in range(NGRP):
        sbuf, dbuf = slabs[g % 2]
        if g == 0:
            pltpu.sync_copy(src_hbm.at[wid, pl.ds(0, G)], sbuf)
            pltpu.sync_copy(dst_hbm.at[wid, pl.ds(0, G2)], dbuf)
        else:
            for c in pend:
                c.wait()
        if g + 1 < NGRP:
            nsb, ndb = slabs[(g + 1) % 2]
            pend = (
                pltpu.async_copy(src_hbm.at[wid, pl.ds((g + 1) * G, G)],
                                 nsb, sem_slab),
                pltpu.async_copy(dst_hbm.at[wid, pl.ds((g + 1) * G2, G2)],
                                 ndb, sem_slab),
            )

        pltpu.async_copy(h_hbm.at[sbuf.at[0]], p0, semA)
        pltpu.async_copy(h_hbm.at[sbuf.at[1]], p1, semB)

        def _one(j, pbuf, sem, sbuf, dbuf):
            pltpu.make_async_copy(h_hbm.at[sbuf.at[j]], pbuf, sem).wait()
            _convert(pbuf, f0, 0)
            pltpu.sync_copy(f0, acc.at[dbuf.at[2 * j]], add=True)
            _convert(pbuf, f1, HALF)
            pltpu.sync_copy(f1, acc.at[dbuf.at[2 * j + 1]], add=True)

            @pl.when(j + 2 < G)
            def _():
                pltpu.async_copy(h_hbm.at[sbuf.at[j + 2]], pbuf, sem)

        def _pair(s, carry, sbuf=sbuf, dbuf=dbuf):
            _one(2 * s, p0, semA, sbuf, dbuf)
            _one(2 * s + 1, p1, semB, sbuf, dbuf)
            return carry

        lax.fori_loop(0, G // 2, _pair, 0)

    plsc.subcore_barrier()

    pltpu.sync_copy(acc.at[pl.ds(sid * ROWS_PT, ROWS_PT)],
                    out_hbm.at[cid, pl.ds(sid * ROWS_PT, ROWS_PT)])


_sc_segment_sum = pl.kernel(
    _sc_segment_sum_body,
    out_type=jax.ShapeDtypeStruct((NC, N_PAD, D), jnp.float32),
    mesh=plsc.VectorSubcoreMesh(core_axis_name="c", subcore_axis_name="s",
                                num_cores=NC),
    compiler_params=pltpu.CompilerParams(use_tc_tiling_on_sc=False,
                                         needs_layout_passes=False),
    scratch_types=[
        pltpu.VMEM((G, CHUNK), jnp.int32),
        pltpu.VMEM((G, CHUNK), jnp.int32),
        pltpu.VMEM((G2, HALF), jnp.int32),
        pltpu.VMEM((G2, HALF), jnp.int32),
        pltpu.VMEM((CHUNK, DP), jnp.int32),
        pltpu.VMEM((CHUNK, DP), jnp.int32),
        pltpu.VMEM((HALF, D), jnp.float32),
        pltpu.VMEM((HALF, D), jnp.float32),
        pltpu.VMEM_SHARED((N_PAD, D), jnp.float32),
        pltpu.SemaphoreType.DMA,
        pltpu.SemaphoreType.DMA,
        pltpu.SemaphoreType.DMA,
    ],
)


BN = 512


def _mlp_body(last, h_ref, p0_ref, p1_ref, w1_ref, b1_ref, w2_ref, b2_ref,
              w2n_ref, b2n_ref, o_ref, opk_ref=None):
    m = h_ref[...] + p0_ref[...] + p1_ref[...]
    t = jnp.maximum(jnp.dot(m, w1_ref[...], preferred_element_type=jnp.float32,
                            precision=lax.Precision.HIGHEST) + b1_ref[...], 0.0)
    o = jnp.dot(t, w2_ref[...], preferred_element_type=jnp.float32,
                precision=lax.Precision.HIGHEST) + b2_ref[...]
    if last:
        nrm = jnp.sqrt(jnp.sum(o * o, axis=1, keepdims=True))
        o_ref[...] = o / jnp.maximum(nrm, 1e-12)
    else:
        o_ref[...] = jnp.maximum(o, 0.0)
        on = jnp.dot(t, w2n_ref[...], preferred_element_type=jnp.float32,
                     precision=lax.Precision.HIGHEST) + b2n_ref[...]
        opk_ref[...] = jnp.maximum(on, 0.0).astype(jnp.bfloat16)


def _mlp(h, p0, p1, w1, b1, w2, b2, w2n, b2n, last):
    row = pl.BlockSpec((BN, D), lambda i: (i, 0))
    full = pl.BlockSpec((D, D), lambda i: (0, 0))
    bias = pl.BlockSpec((1, D), lambda i: (0, 0))
    if last:
        out_specs = row
        out_shape = jax.ShapeDtypeStruct((N_PAD, D), jnp.float32)
    else:
        out_specs = [row, row]
        out_shape = [jax.ShapeDtypeStruct((N_PAD, D), jnp.float32),
                     jax.ShapeDtypeStruct((N_PAD, D), jnp.bfloat16)]
    return pl.pallas_call(
        functools.partial(_mlp_body, last),
        grid=(N_PAD // BN,),
        in_specs=[row, row, row, full, bias, full, bias, full, bias],
        out_specs=out_specs,
        out_shape=out_shape,
    )(h, p0, p1, w1, b1.reshape(1, D), w2, b2.reshape(1, D),
      w2n, b2n.reshape(1, D))


def _pack(hb):
    return lax.bitcast_convert_type(hb.reshape(N_PAD, DP, 2), jnp.int32)


def kernel(x, edge_index, edge_weight, W1_0, b1_0, W2_0, b2_0,
           W1_1, b1_1, W2_1, b2_1):
    pad = E_PAD - E
    src4 = jnp.concatenate([edge_index[0], jnp.zeros((pad,), jnp.int32)]
                           ).reshape(NW, NCH, CHUNK)
    dst_flat = jnp.concatenate([edge_index[1], jnp.full((pad,), N, jnp.int32)])
    dst4 = dst_flat.reshape(NW, 2 * NCH, HALF)
    x_pad = jnp.zeros((N_PAD, D), jnp.float32).at[:N].set(x)

    perm = jnp.asarray(PERM)
    x_perm = x_pad[:, perm]
    W1_0p = W1_0[perm, :]
    W1_1p = W1_1[perm, :]
    W2_0p = W2_0[:, perm]
    b2_0p = b2_0[perm]

    parts = _sc_segment_sum(_pack(x_pad.astype(jnp.bfloat16)), src4, dst4)
    h1p, h1b = _mlp(x_perm, parts[0], parts[1], W1_0p, b1_0,
                    W2_0p, b2_0p, W2_0, b2_0, last=False)
    parts = _sc_segment_sum(_pack(h1b), src4, dst4)
    h2 = _mlp(h1p, parts[0], parts[1], W1_1p, b1_1,
              W2_1, b2_1, W2_1, b2_1, last=True)
    return h2[:N]


# speedup vs baseline: 1.5408x; 1.1046x over previous
import functools

import jax
import jax.numpy as jnp
import numpy as np
from jax import lax
from jax.experimental import pallas as pl
from jax.experimental.pallas import tpu as pltpu
from jax.experimental.pallas import tpu_sc as plsc

N = 10000
D = 128
E = 320000

NC = 2
NS = 16
NW = NC * NS

CHUNK = 128
HALF = CHUNK // 2
DP = D // 2
NCH = 80
G = 16
G2 = 2 * G
EPT = NCH * CHUNK
E_PAD = NW * EPT
N_PAD = 10240
ROWS_PT = N_PAD // NS

PERM = np.concatenate([np.arange(0, D, 2), np.arange(1, D, 2)])


def _sc_segment_sum_body(h_hbm, src_hbm, dst_hbm, out_hbm,
                         sidx0, sidx1, didx0, didx1, p0, p1, f0, f1, acc,
                         semA, semB, sem_slab):
    cid = lax.axis_index("c")
    sid = lax.axis_index("s")
    wid = cid * NS + sid

    def _zrow(i, carry):
        f0[i // 8, pl.ds((i % 8) * 16, 16)] = jnp.zeros((16,), jnp.float32)
        return carry

    lax.fori_loop(0, HALF * 8, _zrow, 0)

    def _zcp(k, carry):
        pltpu.sync_copy(f0, acc.at[pl.ds(sid * ROWS_PT + k * HALF, HALF)])
        return carry

    lax.fori_loop(0, ROWS_PT // HALF, _zcp, 0)
    plsc.subcore_barrier()

    shift16 = jnp.full((16,), 16, jnp.int32)
    himask = jnp.full((16,), -65536, jnp.int32)

    def _convert(pbuf, fbuf, base):
        def _row(r, carry):
            for t in range(4):
                w = pbuf[base + r, pl.ds(16 * t, 16)]
                fbuf[r, pl.ds(16 * t, 16)] = plsc.bitcast(
                    lax.shift_left(w, shift16), jnp.float32)
                fbuf[r, pl.ds(64 + 16 * t, 16)] = plsc.bitcast(
                    lax.bitwise_and(w, himask), jnp.float32)
            return carry

        lax.fori_loop(0, HALF, _row, 0)

    NGRP = NCH // G
    slabs = [(sidx0, didx0), (sidx1, didx1)]
    pend = None
    for g in range(NGRP):
        sbuf, dbuf = slabs[g % 2]
        if g == 0:
            pltpu.sync_copy(src_hbm.at[wid, pl.ds(0, G)], sbuf)
            pltpu.sync_copy(dst_hbm.at[wid, pl.ds(0, G2)], dbuf)
        else:
            for c in pend:
                c.wait()
        if g + 1 < NGRP:
            nsb, ndb = slabs[(g + 1) % 2]
            pend = (
                pltpu.async_copy(src_hbm.at[wid, pl.ds((g + 1) * G, G)],
                                 nsb, sem_slab),
                pltpu.async_copy(dst_hbm.at[wid, pl.ds((g + 1) * G2, G2)],
                                 ndb, sem_slab),
            )

        pltpu.async_copy(h_hbm.at[sbuf.at[0]], p0, semA)
        pltpu.async_copy(h_hbm.at[sbuf.at[1]], p1, semB)

        def _one(j, pbuf, sem, sbuf, dbuf):
            pltpu.make_async_copy(h_hbm.at[sbuf.at[j]], pbuf, sem).wait()
            _convert(pbuf, f0, 0)
            pltpu.sync_copy(f0, acc.at[dbuf.at[2 * j]], add=True)
            _convert(pbuf, f1, HALF)
            pltpu.sync_copy(f1, acc.at[dbuf.at[2 * j + 1]], add=True)

            @pl.when(j + 2 < G)
            def _():
                pltpu.async_copy(h_hbm.at[sbuf.at[j + 2]], pbuf, sem)

        def _pair(s, carry, sbuf=sbuf, dbuf=dbuf):
            _one(2 * s, p0, semA, sbuf, dbuf)
            _one(2 * s + 1, p1, semB, sbuf, dbuf)
            return carry

        lax.fori_loop(0, G // 2, _pair, 0)

    plsc.subcore_barrier()

    pltpu.sync_copy(acc.at[pl.ds(sid * ROWS_PT, ROWS_PT)],
                    out_hbm.at[cid, pl.ds(sid * ROWS_PT, ROWS_PT)])


_sc_segment_sum = pl.kernel(
    _sc_segment_sum_body,
    out_type=jax.ShapeDtypeStruct((NC, N_PAD, D), jnp.float32),
    mesh=plsc.VectorSubcoreMesh(core_axis_name="c", subcore_axis_name="s",
                                num_cores=NC),
    compiler_params=pltpu.CompilerParams(use_tc_tiling_on_sc=False,
                                         needs_layout_passes=False),
    scratch_types=[
        pltpu.VMEM((G, CHUNK), jnp.int32),
        pltpu.VMEM((G, CHUNK), jnp.int32),
        pltpu.VMEM((G2, HALF), jnp.int32),
        pltpu.VMEM((G2, HALF), jnp.int32),
        pltpu.VMEM((CHUNK, DP), jnp.int32),
        pltpu.VMEM((CHUNK, DP), jnp.int32),
        pltpu.VMEM((HALF, D), jnp.float32),
        pltpu.VMEM((HALF, D), jnp.float32),
        pltpu.VMEM_SHARED((N_PAD, D), jnp.float32),
        pltpu.SemaphoreType.DMA,
        pltpu.SemaphoreType.DMA,
        pltpu.SemaphoreType.DMA,
    ],
)


BN = 512


def _pack_rows(o):
    lo = lax.bitcast_convert_type(o[:, :DP].astype(jnp.bfloat16), jnp.uint16)
    hi = lax.bitcast_convert_type(o[:, DP:].astype(jnp.bfloat16), jnp.uint16)
    return lo.astype(jnp.int32) | (hi.astype(jnp.int32) << 16)


def _prep_body(x_ref, pmat_ref, xp_ref, xpk_ref):
    xp = jnp.dot(x_ref[...], pmat_ref[...], preferred_element_type=jnp.float32,
                 precision=lax.Precision.HIGHEST)
    xp_ref[...] = xp
    xpk_ref[...] = _pack_rows(xp)


def _prep(x, pmat):
    row = pl.BlockSpec((BN, D), lambda i: (i, 0))
    return pl.pallas_call(
        _prep_body,
        grid=(N_PAD // BN,),
        in_specs=[row, pl.BlockSpec((D, D), lambda i: (0, 0))],
        out_specs=[row, pl.BlockSpec((BN, DP), lambda i: (i, 0))],
        out_shape=[jax.ShapeDtypeStruct((N_PAD, D), jnp.float32),
                   jax.ShapeDtypeStruct((N_PAD, DP), jnp.int32)],
    )(x, pmat)


def _mlp_body(last, h_ref, parts_ref, w1_ref, b1_ref, w2_ref, b2_ref,
              o_ref, opk_ref=None):
    m = h_ref[...] + parts_ref[0] + parts_ref[1]
    t = jnp.maximum(jnp.dot(m, w1_ref[...], preferred_element_type=jnp.float32,
                            precision=lax.Precision.HIGHEST) + b1_ref[...], 0.0)
    o = jnp.dot(t, w2_ref[...], preferred_element_type=jnp.float32,
                precision=lax.Precision.HIGHEST) + b2_ref[...]
    if last:
        nrm = jnp.sqrt(jnp.sum(o * o, axis=1, keepdims=True))
        o_ref[...] = o / jnp.maximum(nrm, 1e-12)
    else:
        o = jnp.maximum(o, 0.0)
        o_ref[...] = o
        opk_ref[...] = _pack_rows(o)


def _mlp(h, parts, w1, b1, w2, b2, last):
    row = pl.BlockSpec((BN, D), lambda i: (i, 0))
    prow = pl.BlockSpec((2, BN, D), lambda i: (0, i, 0))
    full = pl.BlockSpec((D, D), lambda i: (0, 0))
    bias = pl.BlockSpec((1, D), lambda i: (0, 0))
    if last:
        out_specs = row
        out_shape = jax.ShapeDtypeStruct((N_PAD, D), jnp.float32)
    else:
        out_specs = [row, pl.BlockSpec((BN, DP), lambda i: (i, 0))]
        out_shape = [jax.ShapeDtypeStruct((N_PAD, D), jnp.float32),
                     jax.ShapeDtypeStruct((N_PAD, DP), jnp.int32)]
    return pl.pallas_call(
        functools.partial(_mlp_body, last),
        grid=(N_PAD // BN,),
        in_specs=[row, prow, full, bias, full, bias],
        out_specs=out_specs,
        out_shape=out_shape,
    )(h, parts, w1, b1.reshape(1, D), w2, b2.reshape(1, D))


def kernel(x, edge_index, edge_weight, W1_0, b1_0, W2_0, b2_0,
           W1_1, b1_1, W2_1, b2_1):
    pad = E_PAD - E
    src4 = jnp.concatenate([edge_index[0], jnp.zeros((pad,), jnp.int32)]
                           ).reshape(NW, NCH, CHUNK)
    dst_flat = jnp.concatenate([edge_index[1], jnp.full((pad,), N, jnp.int32)])
    dst4 = dst_flat.reshape(NW, 2 * NCH, HALF)

    perm = jnp.asarray(PERM)
    pmat = jnp.zeros((D, D), jnp.float32).at[perm, jnp.arange(D)].set(1.0)
    W1_0p = W1_0[perm, :]
    W1_1p = W1_1[perm, :]
    W2_0p = W2_0[:, perm]
    b2_0p = b2_0[perm]

    x_perm, x_pack = _prep(x, pmat)
    parts = _sc_segment_sum(x_pack, src4, dst4)
    h1p, h1pk = _mlp(x_perm, parts, W1_0p, b1_0, W2_0p, b2_0p, last=False)
    parts = _sc_segment_sum(h1pk, src4, dst4)
    h2 = _mlp(h1p, parts, W1_1p, b1_1, W2_1, b2_1, last=True)
    return h2[:N]


# speedup vs baseline: 1.6347x; 1.0609x over previous
import functools

import jax
import jax.numpy as jnp
import numpy as np
from jax import lax
from jax.experimental import pallas as pl
from jax.experimental.pallas import tpu as pltpu
from jax.experimental.pallas import tpu_sc as plsc

N = 10000
D = 128
E = 320000

NC = 2
NS = 16
NW = NC * NS

CHUNK = 128
HALF = CHUNK // 2
DP = D // 2
NCH = 80
G = 16
G2 = 2 * G
EPT = NCH * CHUNK
E_PAD = NW * EPT
N_PAD = 10240
ROWS_PT = N_PAD // NS

PERM = np.concatenate([np.arange(0, D, 2), np.arange(1, D, 2)])


def _sc_segment_sum_body(h_hbm, src_hbm, dst_hbm, out_hbm,
                         sidx, didx, p0, p1, f0, acc, semA, semB):
    cid = lax.axis_index("c")
    sid = lax.axis_index("s")
    wid = cid * NS + sid

    def _zrow(i, carry):
        f0[i // 8, pl.ds((i % 8) * 16, 16)] = jnp.zeros((16,), jnp.float32)
        return carry

    lax.fori_loop(0, HALF * 8, _zrow, 0)

    def _zcp(k, carry):
        pltpu.sync_copy(f0, acc.at[pl.ds(sid * ROWS_PT + k * HALF, HALF)])
        return carry

    lax.fori_loop(0, ROWS_PT // HALF, _zcp, 0)
    plsc.subcore_barrier()

    shift16 = jnp.full((16,), 16, jnp.int32)
    himask = jnp.full((16,), -65536, jnp.int32)

    def _convert(pbuf, fbuf, base):
        def _row(r, carry):
            for t in range(4):
                w = pbuf[base + r, pl.ds(16 * t, 16)]
                fbuf[r, pl.ds(16 * t, 16)] = plsc.bitcast(
                    lax.shift_left(w, shift16), jnp.float32)
                fbuf[r, pl.ds(64 + 16 * t, 16)] = plsc.bitcast(
                    lax.bitwise_and(w, himask), jnp.float32)
            return carry

        lax.fori_loop(0, HALF, _row, 0)

    pltpu.sync_copy(src_hbm.at[wid], sidx)
    pltpu.sync_copy(dst_hbm.at[wid], didx)

    pltpu.async_copy(h_hbm.at[sidx.at[0]], p0, semA)
    pltpu.async_copy(h_hbm.at[sidx.at[1]], p1, semB)

    def _one(j, pbuf, sem):
        pltpu.make_async_copy(h_hbm.at[sidx.at[j]], pbuf, sem).wait()
        _convert(pbuf, f0, 0)
        pltpu.sync_copy(f0, acc.at[didx.at[2 * j]], add=True)
        _convert(pbuf, f0, HALF)
        pltpu.sync_copy(f0, acc.at[didx.at[2 * j + 1]], add=True)

        @pl.when(j + 2 < NCH)
        def _():
            pltpu.async_copy(h_hbm.at[sidx.at[j + 2]], pbuf, sem)

    def _pair(s, carry):
        _one(2 * s, p0, semA)
        _one(2 * s + 1, p1, semB)
        return carry

    lax.fori_loop(0, NCH // 2, _pair, 0)
    plsc.subcore_barrier()

    pltpu.sync_copy(acc.at[pl.ds(sid * ROWS_PT, ROWS_PT)],
                    out_hbm.at[cid, pl.ds(sid * ROWS_PT, ROWS_PT)])


_sc_segment_sum = pl.kernel(
    _sc_segment_sum_body,
    out_type=jax.ShapeDtypeStruct((NC, N_PAD, D), jnp.float32),
    mesh=plsc.VectorSubcoreMesh(core_axis_name="c", subcore_axis_name="s",
                                num_cores=NC),
    compiler_params=pltpu.CompilerParams(use_tc_tiling_on_sc=False,
                                         needs_layout_passes=False),
    scratch_types=[
        pltpu.VMEM((NCH, CHUNK), jnp.int32),
        pltpu.VMEM((2 * NCH, HALF), jnp.int32),
        pltpu.VMEM((CHUNK, DP), jnp.int32),
        pltpu.VMEM((CHUNK, DP), jnp.int32),
        pltpu.VMEM((HALF, D), jnp.float32),
        pltpu.VMEM_SHARED((N_PAD, D), jnp.float32),
        pltpu.SemaphoreType.DMA,
        pltpu.SemaphoreType.DMA,
    ],
)


BN = 512


def _pack_rows(o):
    lo = lax.bitcast_convert_type(o[:, :DP].astype(jnp.bfloat16), jnp.uint16)
    hi = lax.bitcast_convert_type(o[:, DP:].astype(jnp.bfloat16), jnp.uint16)
    return lo.astype(jnp.int32) | (hi.astype(jnp.int32) << 16)


def _prep_body(x_ref, pmat_ref, xp_ref, xpk_ref):
    xp = jnp.dot(x_ref[...], pmat_ref[...], preferred_element_type=jnp.float32,
                 precision=lax.Precision.HIGHEST)
    xp_ref[...] = xp
    xpk_ref[...] = _pack_rows(xp)


def _prep(x, pmat):
    row = pl.BlockSpec((BN, D), lambda i: (i, 0))
    return pl.pallas_call(
        _prep_body,
        grid=(N_PAD // BN,),
        in_specs=[row, pl.BlockSpec((D, D), lambda i: (0, 0))],
        out_specs=[row, pl.BlockSpec((BN, DP), lambda i: (i, 0))],
        out_shape=[jax.ShapeDtypeStruct((N_PAD, D), jnp.float32),
                   jax.ShapeDtypeStruct((N_PAD, DP), jnp.int32)],
    )(x, pmat)


def _mlp_body(last, h_ref, parts_ref, w1_ref, b1_ref, w2_ref, b2_ref,
              o_ref, opk_ref=None):
    m = h_ref[...] + parts_ref[0] + parts_ref[1]
    t = jnp.maximum(jnp.dot(m, w1_ref[...], preferred_element_type=jnp.float32,
                            precision=lax.Precision.HIGHEST) + b1_ref[...], 0.0)
    o = jnp.dot(t, w2_ref[...], preferred_element_type=jnp.float32,
                precision=lax.Precision.HIGHEST) + b2_ref[...]
    if last:
        nrm = jnp.sqrt(jnp.sum(o * o, axis=1, keepdims=True))
        o_ref[...] = o / jnp.maximum(nrm, 1e-12)
    else:
        o = jnp.maximum(o, 0.0)
        o_ref[...] = o
        opk_ref[...] = _pack_rows(o)


def _mlp(h, parts, w1, b1, w2, b2, last):
    row = pl.BlockSpec((BN, D), lambda i: (i, 0))
    prow = pl.BlockSpec((2, BN, D), lambda i: (0, i, 0))
    full = pl.BlockSpec((D, D), lambda i: (0, 0))
    bias = pl.BlockSpec((1, D), lambda i: (0, 0))
    if last:
        out_specs = row
        out_shape = jax.ShapeDtypeStruct((N_PAD, D), jnp.float32)
    else:
        out_specs = [row, pl.BlockSpec((BN, DP), lambda i: (i, 0))]
        out_shape = [jax.ShapeDtypeStruct((N_PAD, D), jnp.float32),
                     jax.ShapeDtypeStruct((N_PAD, DP), jnp.int32)]
    return pl.pallas_call(
        functools.partial(_mlp_body, last),
        grid=(N_PAD // BN,),
        in_specs=[row, prow, full, bias, full, bias],
        out_specs=out_specs,
        out_shape=out_shape,
    )(h, parts, w1, b1.reshape(1, D), w2, b2.reshape(1, D))


def kernel(x, edge_index, edge_weight, W1_0, b1_0, W2_0, b2_0,
           W1_1, b1_1, W2_1, b2_1):
    pad = E_PAD - E
    src4 = jnp.concatenate([edge_index[0], jnp.zeros((pad,), jnp.int32)]
                           ).reshape(NW, NCH, CHUNK)
    dst_flat = jnp.concatenate([edge_index[1], jnp.full((pad,), N, jnp.int32)])
    dst4 = dst_flat.reshape(NW, 2 * NCH, HALF)

    perm = jnp.asarray(PERM)
    pmat = jnp.zeros((D, D), jnp.float32).at[perm, jnp.arange(D)].set(1.0)
    W1_0p = W1_0[perm, :]
    W1_1p = W1_1[perm, :]
    W2_0p = W2_0[:, perm]
    b2_0p = b2_0[perm]

    x_perm, x_pack = _prep(x, pmat)
    parts = _sc_segment_sum(x_pack, src4, dst4)
    h1p, h1pk = _mlp(x_perm, parts, W1_0p, b1_0, W2_0p, b2_0p, last=False)
    parts = _sc_segment_sum(h1pk, src4, dst4)
    h2 = _mlp(h1p, parts, W1_1p, b1_1, W2_1, b2_1, last=True)
    return h2[:N]


# speedup vs baseline: 1.7143x; 1.0487x over previous
import functools

import jax
import jax.numpy as jnp
import numpy as np
from jax import lax
from jax.experimental import pallas as pl
from jax.experimental.pallas import tpu as pltpu
from jax.experimental.pallas import tpu_sc as plsc

N = 10000
D = 128
E = 320000

NC = 2
NS = 16
NW = NC * NS

CHUNK = 128
HALF = CHUNK // 2
DP = D // 2
NCH = 80
G = 16
G2 = 2 * G
EPT = NCH * CHUNK
E_PAD = NW * EPT
N_PAD = 10240
ROWS_PT = N_PAD // NS

PERM = np.concatenate([np.arange(0, D, 2), np.arange(1, D, 2)])


def _sc_segment_sum_body(h_hbm, src_hbm, dst_hbm, out_hbm,
                         sidx, didx, p0, p1, f0, acc, semA, semB):
    cid = lax.axis_index("c")
    sid = lax.axis_index("s")
    wid = cid * NS + sid

    def _zrow(i, carry):
        f0[i // 8, pl.ds((i % 8) * 16, 16)] = jnp.zeros((16,), jnp.float32)
        return carry

    lax.fori_loop(0, HALF * 8, _zrow, 0)

    def _zcp(k, carry):
        pltpu.sync_copy(f0, acc.at[pl.ds(sid * ROWS_PT + k * HALF, HALF)])
        return carry

    lax.fori_loop(0, ROWS_PT // HALF, _zcp, 0)
    plsc.subcore_barrier()

    shift16 = jnp.full((16,), 16, jnp.int32)
    himask = jnp.full((16,), -65536, jnp.int32)

    def _convert(pbuf, fbuf, base):
        def _row(r, carry):
            for t in range(4):
                w = pbuf[base + r, pl.ds(16 * t, 16)]
                fbuf[r, pl.ds(16 * t, 16)] = plsc.bitcast(
                    lax.shift_left(w, shift16), jnp.float32)
                fbuf[r, pl.ds(64 + 16 * t, 16)] = plsc.bitcast(
                    lax.bitwise_and(w, himask), jnp.float32)
            return carry

        lax.fori_loop(0, HALF, _row, 0)

    pltpu.sync_copy(src_hbm.at[wid], sidx)
    pltpu.sync_copy(dst_hbm.at[wid], didx)

    pltpu.async_copy(h_hbm.at[sidx.at[0]], p0, semA)
    pltpu.async_copy(h_hbm.at[sidx.at[1]], p1, semB)

    def _one(j, pbuf, sem):
        pltpu.make_async_copy(h_hbm.at[sidx.at[j]], pbuf, sem).wait()
        _convert(pbuf, f0, 0)
        pltpu.sync_copy(f0, acc.at[didx.at[2 * j]], add=True)
        _convert(pbuf, f0, HALF)
        pltpu.sync_copy(f0, acc.at[didx.at[2 * j + 1]], add=True)

        @pl.when(j + 2 < NCH)
        def _():
            pltpu.async_copy(h_hbm.at[sidx.at[j + 2]], pbuf, sem)

    def _pair(s, carry):
        _one(2 * s, p0, semA)
        _one(2 * s + 1, p1, semB)
        return carry

    lax.fori_loop(0, NCH // 2, _pair, 0)
    plsc.subcore_barrier()

    pltpu.sync_copy(acc.at[pl.ds(sid * ROWS_PT, ROWS_PT)],
                    out_hbm.at[cid, pl.ds(sid * ROWS_PT, ROWS_PT)])


_sc_segment_sum = pl.kernel(
    _sc_segment_sum_body,
    out_type=jax.ShapeDtypeStruct((NC, N_PAD, D), jnp.float32),
    mesh=plsc.VectorSubcoreMesh(core_axis_name="c", subcore_axis_name="s",
                                num_cores=NC),
    compiler_params=pltpu.CompilerParams(use_tc_tiling_on_sc=False,
                                         needs_layout_passes=False),
    scratch_types=[
        pltpu.VMEM((NCH, CHUNK), jnp.int32),
        pltpu.VMEM((2 * NCH, HALF), jnp.int32),
        pltpu.VMEM((CHUNK, DP), jnp.int32),
        pltpu.VMEM((CHUNK, DP), jnp.int32),
        pltpu.VMEM((HALF, D), jnp.float32),
        pltpu.VMEM_SHARED((N_PAD, D), jnp.float32),
        pltpu.SemaphoreType.DMA,
        pltpu.SemaphoreType.DMA,
    ],
)


BN = 1024


def _pack_rows(o):
    lo = lax.bitcast_convert_type(o[:, :DP].astype(jnp.bfloat16), jnp.uint16)
    hi = lax.bitcast_convert_type(o[:, DP:].astype(jnp.bfloat16), jnp.uint16)
    return lo.astype(jnp.int32) | (hi.astype(jnp.int32) << 16)


def _prep_body(x_ref, pmat_ref, xp_ref, xpk_ref):
    xp = jnp.dot(x_ref[...], pmat_ref[...], preferred_element_type=jnp.float32,
                 precision=lax.Precision.HIGHEST)
    xp_ref[...] = xp
    xpk_ref[...] = _pack_rows(xp)


def _prep(x, pmat):
    row = pl.BlockSpec((BN, D), lambda i: (i, 0))
    return pl.pallas_call(
        _prep_body,
        grid=(N_PAD // BN,),
        in_specs=[row, pl.BlockSpec((D, D), lambda i: (0, 0))],
        out_specs=[row, pl.BlockSpec((BN, DP), lambda i: (i, 0))],
        out_shape=[jax.ShapeDtypeStruct((N_PAD, D), jnp.float32),
                   jax.ShapeDtypeStruct((N_PAD, DP), jnp.int32)],
    )(x, pmat)


def _mlp_body(last, h_ref, parts_ref, w1_ref, b1_ref, w2_ref, b2_ref,
              o_ref, opk_ref=None):
    m = h_ref[...] + parts_ref[0] + parts_ref[1]
    t = jnp.maximum(jnp.dot(m, w1_ref[...], preferred_element_type=jnp.float32,
                            precision=lax.Precision.HIGHEST) + b1_ref[...], 0.0)
    o = jnp.dot(t, w2_ref[...], preferred_element_type=jnp.float32,
                precision=lax.Precision.HIGHEST) + b2_ref[...]
    if last:
        nrm = jnp.sqrt(jnp.sum(o * o, axis=1, keepdims=True))
        o_ref[...] = o / jnp.maximum(nrm, 1e-12)
    else:
        o = jnp.maximum(o, 0.0)
        o_ref[...] = o
        opk_ref[...] = _pack_rows(o)


def _mlp(h, parts, w1, b1, w2, b2, last):
    row = pl.BlockSpec((BN, D), lambda i: (i, 0))
    prow = pl.BlockSpec((2, BN, D), lambda i: (0, i, 0))
    full = pl.BlockSpec((D, D), lambda i: (0, 0))
    bias = pl.BlockSpec((1, D), lambda i: (0, 0))
    if last:
        out_specs = row
        out_shape = jax.ShapeDtypeStruct((N, D), jnp.float32)
    else:
        out_specs = [row, pl.BlockSpec((BN, DP), lambda i: (i, 0))]
        out_shape = [jax.ShapeDtypeStruct((N_PAD, D), jnp.float32),
                     jax.ShapeDtypeStruct((N_PAD, DP), jnp.int32)]
    return pl.pallas_call(
        functools.partial(_mlp_body, last),
        grid=(N_PAD // BN,),
        in_specs=[row, prow, full, bias, full, bias],
        out_specs=out_specs,
        out_shape=out_shape,
    )(h, parts, w1, b1.reshape(1, D), w2, b2.reshape(1, D))


def kernel(x, edge_index, edge_weight, W1_0, b1_0, W2_0, b2_0,
           W1_1, b1_1, W2_1, b2_1):
    ei = jnp.pad(edge_index, ((0, 0), (0, E_PAD - E)), constant_values=N)
    src4 = ei[0].reshape(NW, NCH, CHUNK)
    dst4 = ei[1].reshape(NW, 2 * NCH, HALF)

    perm = jnp.asarray(PERM)
    pmat = jnp.zeros((D, D), jnp.float32).at[perm, jnp.arange(D)].set(1.0)
    W1_0p = W1_0[perm, :]
    W1_1p = W1_1[perm, :]
    W2_0p = W2_0[:, perm]
    b2_0p = b2_0[perm]

    x_perm, x_pack = _prep(x, pmat)
    parts = _sc_segment_sum(x_pack, src4, dst4)
    h1p, h1pk = _mlp(x_perm, parts, W1_0p, b1_0, W2_0p, b2_0p, last=False)
    parts = _sc_segment_sum(h1pk, src4, dst4)
    return _mlp(h1p, parts, W1_1p, b1_1, W2_1, b2_1, last=True)


# speedup vs baseline: 1.7276x; 1.0078x over previous
import functools

import jax
import jax.numpy as jnp
import numpy as np
from jax import lax
from jax.experimental import pallas as pl
from jax.experimental.pallas import tpu as pltpu
from jax.experimental.pallas import tpu_sc as plsc

N = 10000
D = 128
E = 320000

NC = 2
NS = 16
NW = NC * NS

CHUNK = 128
HALF = CHUNK // 2
DP = D // 2
NCH = 80
G = 16
G2 = 2 * G
EPT = NCH * CHUNK
E_PAD = NW * EPT
N_PAD = 10240
ROWS_PT = N_PAD // NS

PERM = np.concatenate([np.arange(0, D, 2), np.arange(1, D, 2)])


def _sc_segment_sum_body(h_hbm, src_hbm, dst_hbm, out_hbm,
                         sidx, didx, p0, p1, f0, acc, semA, semB):
    cid = lax.axis_index("c")
    sid = lax.axis_index("s")
    wid = cid * NS + sid

    pltpu.sync_copy(src_hbm.at[wid], sidx)
    pltpu.async_copy(h_hbm.at[sidx.at[0]], p0, semA)
    pltpu.async_copy(h_hbm.at[sidx.at[1]], p1, semB)
    pltpu.sync_copy(dst_hbm.at[wid], didx)

    def _zrow(i, carry):
        f0[i // 8, pl.ds((i % 8) * 16, 16)] = jnp.zeros((16,), jnp.float32)
        return carry

    lax.fori_loop(0, HALF * 8, _zrow, 0)

    def _zcp(k, carry):
        pltpu.sync_copy(f0, acc.at[pl.ds(sid * ROWS_PT + k * HALF, HALF)])
        return carry

    lax.fori_loop(0, ROWS_PT // HALF, _zcp, 0)
    plsc.subcore_barrier()

    shift16 = jnp.full((16,), 16, jnp.int32)
    himask = jnp.full((16,), -65536, jnp.int32)

    def _convert(pbuf, fbuf, base):
        def _row(r, carry):
            for t in range(4):
                w = pbuf[base + r, pl.ds(16 * t, 16)]
                fbuf[r, pl.ds(16 * t, 16)] = plsc.bitcast(
                    lax.shift_left(w, shift16), jnp.float32)
                fbuf[r, pl.ds(64 + 16 * t, 16)] = plsc.bitcast(
                    lax.bitwise_and(w, himask), jnp.float32)
            return carry

        lax.fori_loop(0, HALF, _row, 0)

    def _one(j, pbuf, sem):
        pltpu.make_async_copy(h_hbm.at[sidx.at[j]], pbuf, sem).wait()
        _convert(pbuf, f0, 0)
        pltpu.sync_copy(f0, acc.at[didx.at[2 * j]], add=True)
        _convert(pbuf, f0, HALF)
        pltpu.sync_copy(f0, acc.at[didx.at[2 * j + 1]], add=True)

        @pl.when(j + 2 < NCH)
        def _():
            pltpu.async_copy(h_hbm.at[sidx.at[j + 2]], pbuf, sem)

    def _pair(s, carry):
        _one(2 * s, p0, semA)
        _one(2 * s + 1, p1, semB)
        return carry

    lax.fori_loop(0, NCH // 2, _pair, 0)
    plsc.subcore_barrier()

    pltpu.sync_copy(acc.at[pl.ds(sid * ROWS_PT, ROWS_PT)],
                    out_hbm.at[cid, pl.ds(sid * ROWS_PT, ROWS_PT)])


_sc_segment_sum = pl.kernel(
    _sc_segment_sum_body,
    out_type=jax.ShapeDtypeStruct((NC, N_PAD, D), jnp.float32),
    mesh=plsc.VectorSubcoreMesh(core_axis_name="c", subcore_axis_name="s",
                                num_cores=NC),
    compiler_params=pltpu.CompilerParams(use_tc_tiling_on_sc=False,
                                         needs_layout_passes=False),
    scratch_types=[
        pltpu.VMEM((NCH, CHUNK), jnp.int32),
        pltpu.VMEM((2 * NCH, HALF), jnp.int32),
        pltpu.VMEM((CHUNK, DP), jnp.int32),
        pltpu.VMEM((CHUNK, DP), jnp.int32),
        pltpu.VMEM((HALF, D), jnp.float32),
        pltpu.VMEM_SHARED((N_PAD, D), jnp.float32),
        pltpu.SemaphoreType.DMA,
        pltpu.SemaphoreType.DMA,
    ],
)


BN = 1024


def _pack_rows(o):
    lo = lax.bitcast_convert_type(o[:, :DP].astype(jnp.bfloat16), jnp.uint16)
    hi = lax.bitcast_convert_type(o[:, DP:].astype(jnp.bfloat16), jnp.uint16)
    return lo.astype(jnp.int32) | (hi.astype(jnp.int32) << 16)


def _prep_body(x_ref, pmat_ref, xp_ref, xpk_ref):
    xp = jnp.dot(x_ref[...], pmat_ref[...], preferred_element_type=jnp.float32,
                 precision=lax.Precision.HIGHEST)
    xp_ref[...] = xp
    xpk_ref[...] = _pack_rows(xp)


def _prep(x, pmat):
    row = pl.BlockSpec((BN, D), lambda i: (i, 0))
    return pl.pallas_call(
        _prep_body,
        grid=(N_PAD // BN,),
        in_specs=[row, pl.BlockSpec((D, D), lambda i: (0, 0))],
        out_specs=[row, pl.BlockSpec((BN, DP), lambda i: (i, 0))],
        out_shape=[jax.ShapeDtypeStruct((N_PAD, D), jnp.float32),
                   jax.ShapeDtypeStruct((N_PAD, DP), jnp.int32)],
    )(x, pmat)


def _mlp_body(last, h_ref, parts_ref, w1_ref, b1_ref, w2_ref, b2_ref,
              o_ref, opk_ref=None):
    m = h_ref[...] + parts_ref[0] + parts_ref[1]
    t = jnp.maximum(jnp.dot(m, w1_ref[...], preferred_element_type=jnp.float32,
                            precision=lax.Precision.HIGHEST) + b1_ref[...], 0.0)
    o = jnp.dot(t, w2_ref[...], preferred_element_type=jnp.float32,
                precision=lax.Precision.HIGHEST) + b2_ref[...]
    if last:
        nrm = jnp.sqrt(jnp.sum(o * o, axis=1, keepdims=True))
        o_ref[...] = o / jnp.maximum(nrm, 1e-12)
    else:
        o = jnp.maximum(o, 0.0)
        o_ref[...] = o
        opk_ref[...] = _pack_rows(o)


def _mlp(h, parts, w1, b1, w2, b2, last):
    row = pl.BlockSpec((BN, D), lambda i: (i, 0))
    prow = pl.BlockSpec((2, BN, D), lambda i: (0, i, 0))
    full = pl.BlockSpec((D, D), lambda i: (0, 0))
    bias = pl.BlockSpec((1, D), lambda i: (0, 0))
    if last:
        out_specs = row
        out_shape = jax.ShapeDtypeStruct((N, D), jnp.float32)
    else:
        out_specs = [row, pl.BlockSpec((BN, DP), lambda i: (i, 0))]
        out_shape = [jax.ShapeDtypeStruct((N_PAD, D), jnp.float32),
                     jax.ShapeDtypeStruct((N_PAD, DP), jnp.int32)]
    return pl.pallas_call(
        functools.partial(_mlp_body, last),
        grid=(N_PAD // BN,),
        in_specs=[row, prow, full, bias, full, bias],
        out_specs=out_specs,
        out_shape=out_shape,
    )(h, parts, w1, b1.reshape(1, D), w2, b2.reshape(1, D))


def kernel(x, edge_index, edge_weight, W1_0, b1_0, W2_0, b2_0,
           W1_1, b1_1, W2_1, b2_1):
    ei = jnp.pad(edge_index, ((0, 0), (0, E_PAD - E)), constant_values=N)
    src4 = ei[0].reshape(NW, NCH, CHUNK)
    dst4 = ei[1].reshape(NW, 2 * NCH, HALF)

    perm = jnp.asarray(PERM)
    pmat = jnp.zeros((D, D), jnp.float32).at[perm, jnp.arange(D)].set(1.0)
    W1_0p = W1_0[perm, :]
    W1_1p = W1_1[perm, :]
    W2_0p = W2_0[:, perm]
    b2_0p = b2_0[perm]

    x_perm, x_pack = _prep(x, pmat)
    parts = _sc_segment_sum(x_pack, src4, dst4)
    h1p, h1pk = _mlp(x_perm, parts, W1_0p, b1_0, W2_0p, b2_0p, last=False)
    parts = _sc_segment_sum(h1pk, src4, dst4)
    return _mlp(h1p, parts, W1_1p, b1_1, W2_1, b2_1, last=True)
